# baseline (speedup 1.0000x reference)
import functools

import jax
import jax.numpy as jnp
from jax import lax
from jax.experimental import pallas as pl
from jax.experimental.pallas import tpu as pltpu
from jax.experimental.pallas import tpu_sc as plsc

_NC = 2
_NS = 16
_NW = _NC * _NS
_B = 128

_SC_PARAMS = pltpu.CompilerParams(use_tc_tiling_on_sc=False)


def _sc_mesh():
    return plsc.VectorSubcoreMesh(core_axis_name="c", subcore_axis_name="s",
                                  num_cores=_NC, num_subcores=_NS)


def _pad_count(count):
    unit = _NW * _B * 2
    return ((count + unit - 1) // unit) * unit


def _seg_sum_sc(table, src3, dst3, zeros_d):
    n, d = table.shape
    _, steps, b = src3.shape
    npt = n // _NS

    @functools.partial(
        pl.kernel,
        out_type=jax.ShapeDtypeStruct((_NC, n, d), jnp.float32),
        mesh=_sc_mesh(),
        scratch_types=[
            pltpu.VMEM((steps, b), jnp.int32),
            pltpu.VMEM((steps, b), jnp.int32),
            pltpu.VMEM((b, d), jnp.float32),
            pltpu.VMEM((b, d), jnp.float32),
            pltpu.VMEM_SHARED((n, d), jnp.float32),
            pltpu.SemaphoreType.DMA,
            pltpu.SemaphoreType.DMA,
        ],
        compiler_params=_SC_PARAMS,
    )
    def k(table_hbm, src_hbm, dst_hbm, zd_hbm, seg_out,
          isa, ida, buf0, buf1, tab_sh, g0, g1):
        cid = lax.axis_index("c")
        sid = lax.axis_index("s")
        wid = cid * _NS + sid
        r0 = sid * npt
        pltpu.sync_copy(zd_hbm.at[pl.ds(r0, npt)], tab_sh.at[pl.ds(r0, npt)])
        pltpu.sync_copy(src_hbm.at[wid], isa)
        pltpu.sync_copy(dst_hbm.at[wid], ida)
        plsc.subcore_barrier()

        pltpu.async_copy(table_hbm.at[isa.at[0]], buf0, g0)
        pltpu.async_copy(table_hbm.at[isa.at[1]], buf1, g1)

        def body(i, carry):
            j0 = 2 * i
            j1 = j0 + 1
            pltpu.make_async_copy(table_hbm.at[isa.at[j0]], buf0, g0).wait()
            pltpu.sync_copy(buf0, tab_sh.at[ida.at[j0]], add=True)

            @pl.when(j0 + 2 < steps)
            def _():
                pltpu.async_copy(table_hbm.at[isa.at[j0 + 2]], buf0, g0)

            pltpu.make_async_copy(table_hbm.at[isa.at[j1]], buf1, g1).wait()
            pltpu.sync_copy(buf1, tab_sh.at[ida.at[j1]], add=True)

            @pl.when(j1 + 2 < steps)
            def _():
                pltpu.async_copy(table_hbm.at[isa.at[j1 + 2]], buf1, g1)

            return carry

        lax.fori_loop(0, steps // 2, body, 0)
        plsc.subcore_barrier()
        pltpu.sync_copy(tab_sh.at[pl.ds(r0, npt)],
                        seg_out.at[cid, pl.ds(r0, npt)])

    return k(table, src3, dst3, zeros_d)


def _pair_gather_sc(z, u3, v3, pp):
    n, d = z.shape
    _, steps, b = u3.shape
    ppw = steps * b

    @functools.partial(
        pl.kernel,
        out_type=[jax.ShapeDtypeStruct((pp, d), jnp.float32),
                  jax.ShapeDtypeStruct((pp, d), jnp.float32)],
        mesh=_sc_mesh(),
        scratch_types=[
            pltpu.VMEM((steps, b), jnp.int32),
            pltpu.VMEM((steps, b), jnp.int32),
            pltpu.VMEM((b, d), jnp.float32),
            pltpu.VMEM((b, d), jnp.float32),
            pltpu.VMEM((b, d), jnp.float32),
            pltpu.VMEM((b, d), jnp.float32),
            pltpu.SemaphoreType.DMA,
            pltpu.SemaphoreType.DMA,
            pltpu.SemaphoreType.DMA,
            pltpu.SemaphoreType.DMA,
        ],
        compiler_params=_SC_PARAMS,
    )
    def k(z_hbm, u_hbm, v_hbm, zu_out, zv_out,
          iua, iva, ru0, ru1, rv0, rv1, gu0, gu1, gv0, gv1):
        cid = lax.axis_index("c")
        sid = lax.axis_index("s")
        wid = cid * _NS + sid
        base0 = wid * ppw
        pltpu.sync_copy(u_hbm.at[wid], iua)
        pltpu.sync_copy(v_hbm.at[wid], iva)

        pltpu.async_copy(z_hbm.at[iua.at[0]], ru0, gu0)
        pltpu.async_copy(z_hbm.at[iva.at[0]], rv0, gv0)
        pltpu.async_copy(z_hbm.at[iua.at[1]], ru1, gu1)
        pltpu.async_copy(z_hbm.at[iva.at[1]], rv1, gv1)

        def body(i, carry):
            j0 = 2 * i
            j1 = j0 + 1
            pltpu.make_async_copy(z_hbm.at[iua.at[j0]], ru0, gu0).wait()
            pltpu.sync_copy(ru0, zu_out.at[pl.ds(base0 + j0 * b, b)])
            pltpu.make_async_copy(z_hbm.at[iva.at[j0]], rv0, gv0).wait()
            pltpu.sync_copy(rv0, zv_out.at[pl.ds(base0 + j0 * b, b)])

            @pl.when(j0 + 2 < steps)
            def _():
                pltpu.async_copy(z_hbm.at[iua.at[j0 + 2]], ru0, gu0)
                pltpu.async_copy(z_hbm.at[iva.at[j0 + 2]], rv0, gv0)

            pltpu.make_async_copy(z_hbm.at[iua.at[j1]], ru1, gu1).wait()
            pltpu.sync_copy(ru1, zu_out.at[pl.ds(base0 + j1 * b, b)])
            pltpu.make_async_copy(z_hbm.at[iva.at[j1]], rv1, gv1).wait()
            pltpu.sync_copy(rv1, zv_out.at[pl.ds(base0 + j1 * b, b)])

            @pl.when(j1 + 2 < steps)
            def _():
                pltpu.async_copy(z_hbm.at[iua.at[j1 + 2]], ru1, gu1)
                pltpu.async_copy(z_hbm.at[iva.at[j1 + 2]], rv1, gv1)

            return carry

        lax.fori_loop(0, steps // 2, body, 0)

    return k(z, u3, v3)


def _encode_in(x, wcat):
    n = x.shape[0]
    h2 = wcat.shape[1]
    h = h2 // 2

    def body(x_ref, w_ref, xl_ref, xr_ref):
        xw = jnp.dot(x_ref[...], w_ref[...], preferred_element_type=jnp.float32)
        xl_ref[:, :h] = xw[:, :h]
        xl_ref[:, h:] = jnp.ones((n, 16), jnp.float32)
        xr_ref[...] = xw[:, h:]

    return pl.pallas_call(
        body,
        out_shape=[jax.ShapeDtypeStruct((n, h + 16), jnp.float32),
                   jax.ShapeDtypeStruct((n, h), jnp.float32)],
    )(x, wcat)


def _mid(seg1p, xr, b1r, wcat2):
    n, h = xr.shape
    o2 = wcat2.shape[1]
    o = o2 // 2

    def body(s_ref, xr_ref, b1_ref, w_ref, zl_ref, zr_ref, inv_ref):
        sp = s_ref[...]
        deg = sp[0, :, h:h + 1] + sp[1, :, h:h + 1]
        inv = 1.0 / jnp.maximum(deg, 1.0)
        seg = sp[0, :, :h] + sp[1, :, :h]
        z1 = jnp.maximum(seg * inv + b1_ref[...] + xr_ref[...], 0.0)
        zw = jnp.dot(z1, w_ref[...], preferred_element_type=jnp.float32)
        zl_ref[...] = zw[:, :o]
        zr_ref[...] = zw[:, o:]
        inv_ref[...] = inv

    return pl.pallas_call(
        body,
        out_shape=[jax.ShapeDtypeStruct((n, o), jnp.float32),
                   jax.ShapeDtypeStruct((n, o), jnp.float32),
                   jax.ShapeDtypeStruct((n, 1), jnp.float32)],
    )(seg1p, xr, b1r, wcat2)


def _final_nodes(seg2p, inv, zr, b2r):
    n, o = zr.shape

    def body(s_ref, i_ref, zr_ref, b2_ref, z_ref):
        sp = s_ref[...]
        z_ref[...] = (sp[0] + sp[1]) * i_ref[...] + b2_ref[...] + zr_ref[...]

    return pl.pallas_call(
        body,
        out_shape=jax.ShapeDtypeStruct((n, o), jnp.float32),
    )(seg2p, inv, zr, b2r)


def _mlp(zu, zv, pfp, w1s, wpf, b1m, w2t, b2m, w3p, b3p, bp=2048):
    pp, o = zu.shape
    pfd = pfp.shape[1]
    mh = w1s.shape[1]
    mh2 = w2t.shape[1]
    ow = w3p.shape[1]
    grid = pp // bp

    def body(zu_ref, zv_ref, pf_ref, w1_ref, wp_ref, b1_ref, w2_ref, b2_ref,
             w3_ref, b3_ref, out_ref):
        a = zu_ref[...]
        bv = zv_ref[...]
        ad = jnp.abs(a - bv)
        pr = a * bv
        h1 = (jnp.dot(a, w1_ref[0:o], preferred_element_type=jnp.float32)
              + jnp.dot(bv, w1_ref[o:2 * o], preferred_element_type=jnp.float32)
              + jnp.dot(ad, w1_ref[2 * o:3 * o], preferred_element_type=jnp.float32)
              + jnp.dot(pr, w1_ref[3 * o:4 * o], preferred_element_type=jnp.float32)
              + jnp.dot(pf_ref[...], wp_ref[...], preferred_element_type=jnp.float32)
              + b1_ref[...])
        h1 = jnp.maximum(h1, 0.0)
        h2 = jnp.maximum(jnp.dot(h1, w2_ref[...], preferred_element_type=jnp.float32)
                         + b2_ref[...], 0.0)
        out_ref[...] = jnp.dot(h2, w3_ref[...], preferred_element_type=jnp.float32) + b3_ref[...]

    return pl.pallas_call(
        body,
        grid=(grid,),
        in_specs=[
            pl.BlockSpec((bp, o), lambda i: (i, 0)),
            pl.BlockSpec((bp, o), lambda i: (i, 0)),
            pl.BlockSpec((bp, pfd), lambda i: (i, 0)),
            pl.BlockSpec((4 * o, mh), lambda i: (0, 0)),
            pl.BlockSpec((pfd, mh), lambda i: (0, 0)),
            pl.BlockSpec((1, mh), lambda i: (0, 0)),
            pl.BlockSpec((mh, mh2), lambda i: (0, 0)),
            pl.BlockSpec((1, mh2), lambda i: (0, 0)),
            pl.BlockSpec((mh2, ow), lambda i: (0, 0)),
            pl.BlockSpec((1, ow), lambda i: (0, 0)),
        ],
        out_specs=pl.BlockSpec((bp, ow), lambda i: (i, 0)),
        out_shape=jax.ShapeDtypeStruct((pp, ow), jnp.float32),
    )(zu, zv, pfp, w1s, wpf, b1m, w2t, b2m, w3p, b3p)


def kernel(x, edge_index, edge_label_index, pair_feats,
           Wl1, Wr1, b1, Wl2, Wr2, b2, Wm1, bm1, Wm2, bm2, Wm3, bm3):
    n = x.shape[0]
    e = edge_index.shape[1]
    p = edge_label_index.shape[1]
    h = Wl1.shape[0]
    o = Wl2.shape[0]
    mh = Wm1.shape[0]
    mh2 = Wm2.shape[0]
    pfd = pair_feats.shape[1]

    nunit = _NS * 8
    n_pad = ((n + nunit) // nunit) * nunit
    xp = jnp.pad(x, ((0, n_pad - n), (0, 0)))

    e_pad = _pad_count(e)
    steps_e = e_pad // (_NW * _B)
    src3 = jnp.pad(edge_index[0], (0, e_pad - e), constant_values=n)
    src3 = src3.reshape(_NW, steps_e, _B)
    dst3 = jnp.pad(edge_index[1], (0, e_pad - e), constant_values=n)
    dst3 = dst3.reshape(_NW, steps_e, _B)

    wcat1 = jnp.concatenate([Wl1.T, Wr1.T], axis=1)
    xlaug, xr = _encode_in(xp, wcat1)

    zeros_h = jnp.zeros((n_pad, h + 16), jnp.float32)
    seg1p = _seg_sum_sc(xlaug, src3, dst3, zeros_h)

    wcat2 = jnp.concatenate([Wl2.T, Wr2.T], axis=1)
    zl, zr, inv = _mid(seg1p, xr, b1.reshape(1, h), wcat2)

    zeros_o = jnp.zeros((n_pad, o), jnp.float32)
    seg2p = _seg_sum_sc(zl, src3, dst3, zeros_o)

    z = _final_nodes(seg2p, inv, zr, b2.reshape(1, o))

    pp = _pad_count(p)
    steps_p = pp // (_NW * _B)
    u3 = jnp.pad(edge_label_index[0], (0, pp - p)).reshape(_NW, steps_p, _B)
    v3 = jnp.pad(edge_label_index[1], (0, pp - p)).reshape(_NW, steps_p, _B)
    zu, zv = _pair_gather_sc(z, u3, v3, pp)

    pfpad = 8
    pfp = jnp.pad(pair_feats, ((0, pp - p), (0, pfpad - pfd)))
    w1s = Wm1.T[:4 * o]
    wpf = jnp.pad(Wm1.T[4 * o:], ((0, pfpad - pfd), (0, 0)))
    w3p = jnp.pad(Wm3.T, ((0, 0), (0, 7)))
    b3p = jnp.pad(bm3.reshape(1, 1), ((0, 0), (0, 7)))
    out8 = _mlp(zu, zv, pfp, w1s, wpf, bm1.reshape(1, mh), Wm2.T,
                bm2.reshape(1, mh2), w3p, b3p)
    return out8[:p, 0]


# speedup vs baseline: 1.8361x; 1.8361x over previous
import functools

import jax
import jax.numpy as jnp
from jax import lax
from jax.experimental import pallas as pl
from jax.experimental.pallas import tpu as pltpu
from jax.experimental.pallas import tpu_sc as plsc

_NC = 2
_NS = 16
_NW = _NC * _NS
_B = 128

_SC_PARAMS = pltpu.CompilerParams(use_tc_tiling_on_sc=False)


def _sc_mesh():
    return plsc.VectorSubcoreMesh(core_axis_name="c", subcore_axis_name="s",
                                  num_cores=_NC, num_subcores=_NS)


def _shard_indices(idx, pad_base, pad_count):
    e = idx.shape[0]
    epw = -(-e // _NW)
    if _NW * epw > e:
        tail = pad_base + (jnp.arange(_NW * epw - e, dtype=jnp.int32) % pad_count)
        idx = jnp.concatenate([idx, tail])
    idx2 = idx.reshape(_NW, epw)
    epw_pad = -(-epw // (2 * _B)) * (2 * _B)
    if epw_pad > epw:
        junk = pad_base + (jnp.arange(epw_pad - epw, dtype=jnp.int32) % pad_count)
        idx2 = jnp.concatenate([idx2, jnp.tile(junk[None], (_NW, 1))], axis=1)
    return idx2.reshape(_NW, epw_pad // _B, _B), epw


def _seg_sum_sc(table, src3, dst3, zeros_d):
    n, d = table.shape
    _, steps, b = src3.shape
    npt = n // _NS

    @functools.partial(
        pl.kernel,
        out_type=jax.ShapeDtypeStruct((_NC, n, d), jnp.float32),
        mesh=_sc_mesh(),
        scratch_types=[
            pltpu.VMEM((steps, b), jnp.int32),
            pltpu.VMEM((steps, b), jnp.int32),
            pltpu.VMEM((b, d), jnp.float32),
            pltpu.VMEM((b, d), jnp.float32),
            pltpu.VMEM_SHARED((n, d), jnp.float32),
            pltpu.SemaphoreType.DMA,
            pltpu.SemaphoreType.DMA,
        ],
        compiler_params=_SC_PARAMS,
    )
    def k(table_hbm, src_hbm, dst_hbm, zd_hbm, seg_out,
          isa, ida, buf0, buf1, tab_sh, g0, g1):
        cid = lax.axis_index("c")
        sid = lax.axis_index("s")
        wid = cid * _NS + sid
        r0 = sid * npt
        pltpu.sync_copy(zd_hbm.at[pl.ds(r0, npt)], tab_sh.at[pl.ds(r0, npt)])
        pltpu.sync_copy(src_hbm.at[wid], isa)
        pltpu.sync_copy(dst_hbm.at[wid], ida)
        plsc.subcore_barrier()

        pltpu.async_copy(table_hbm.at[isa.at[0]], buf0, g0)
        pltpu.async_copy(table_hbm.at[isa.at[1]], buf1, g1)

        def body(i, carry):
            j0 = 2 * i
            j1 = j0 + 1
            pltpu.make_async_copy(table_hbm.at[isa.at[j0]], buf0, g0).wait()
            pltpu.sync_copy(buf0, tab_sh.at[ida.at[j0]], add=True)

            @pl.when(j0 + 2 < steps)
            def _():
                pltpu.async_copy(table_hbm.at[isa.at[j0 + 2]], buf0, g0)

            pltpu.make_async_copy(table_hbm.at[isa.at[j1]], buf1, g1).wait()
            pltpu.sync_copy(buf1, tab_sh.at[ida.at[j1]], add=True)

            @pl.when(j1 + 2 < steps)
            def _():
                pltpu.async_copy(table_hbm.at[isa.at[j1 + 2]], buf1, g1)

            return carry

        lax.fori_loop(0, steps // 2, body, 0)
        plsc.subcore_barrier()
        pltpu.sync_copy(tab_sh.at[pl.ds(r0, npt)],
                        seg_out.at[cid, pl.ds(r0, npt)])

    return k(table, src3, dst3, zeros_d)


def _pair_gather_sc(z, u3, v3, pp):
    n, d = z.shape
    _, steps, b = u3.shape
    ppw = steps * b

    @functools.partial(
        pl.kernel,
        out_type=[jax.ShapeDtypeStruct((pp, d), jnp.float32),
                  jax.ShapeDtypeStruct((pp, d), jnp.float32)],
        mesh=_sc_mesh(),
        scratch_types=[
            pltpu.VMEM((steps, b), jnp.int32),
            pltpu.VMEM((steps, b), jnp.int32),
            pltpu.VMEM((b, d), jnp.float32),
            pltpu.VMEM((b, d), jnp.float32),
            pltpu.VMEM((b, d), jnp.float32),
            pltpu.VMEM((b, d), jnp.float32),
            pltpu.SemaphoreType.DMA,
            pltpu.SemaphoreType.DMA,
            pltpu.SemaphoreType.DMA,
            pltpu.SemaphoreType.DMA,
        ],
        compiler_params=_SC_PARAMS,
    )
    def k(z_hbm, u_hbm, v_hbm, zu_out, zv_out,
          iua, iva, ru0, ru1, rv0, rv1, gu0, gu1, gv0, gv1):
        cid = lax.axis_index("c")
        sid = lax.axis_index("s")
        wid = cid * _NS + sid
        base0 = wid * ppw
        pltpu.sync_copy(u_hbm.at[wid], iua)
        pltpu.sync_copy(v_hbm.at[wid], iva)

        pltpu.async_copy(z_hbm.at[iua.at[0]], ru0, gu0)
        pltpu.async_copy(z_hbm.at[iva.at[0]], rv0, gv0)
        pltpu.async_copy(z_hbm.at[iua.at[1]], ru1, gu1)
        pltpu.async_copy(z_hbm.at[iva.at[1]], rv1, gv1)

        def body(i, carry):
            j0 = 2 * i
            j1 = j0 + 1
            pltpu.make_async_copy(z_hbm.at[iua.at[j0]], ru0, gu0).wait()
            pltpu.sync_copy(ru0, zu_out.at[pl.ds(base0 + j0 * b, b)])
            pltpu.make_async_copy(z_hbm.at[iva.at[j0]], rv0, gv0).wait()
            pltpu.sync_copy(rv0, zv_out.at[pl.ds(base0 + j0 * b, b)])

            @pl.when(j0 + 2 < steps)
            def _():
                pltpu.async_copy(z_hbm.at[iua.at[j0 + 2]], ru0, gu0)
                pltpu.async_copy(z_hbm.at[iva.at[j0 + 2]], rv0, gv0)

            pltpu.make_async_copy(z_hbm.at[iua.at[j1]], ru1, gu1).wait()
            pltpu.sync_copy(ru1, zu_out.at[pl.ds(base0 + j1 * b, b)])
            pltpu.make_async_copy(z_hbm.at[iva.at[j1]], rv1, gv1).wait()
            pltpu.sync_copy(rv1, zv_out.at[pl.ds(base0 + j1 * b, b)])

            @pl.when(j1 + 2 < steps)
            def _():
                pltpu.async_copy(z_hbm.at[iua.at[j1 + 2]], ru1, gu1)
                pltpu.async_copy(z_hbm.at[iva.at[j1 + 2]], rv1, gv1)

            return carry

        lax.fori_loop(0, steps // 2, body, 0)

    return k(z, u3, v3)


def _encode_in(x, wcat):
    n = x.shape[0]
    h2 = wcat.shape[1]
    h = h2 // 2

    def body(x_ref, w_ref, xl_ref, xr_ref):
        xw = jnp.dot(x_ref[...], w_ref[...], preferred_element_type=jnp.float32)
        xl_ref[:, :h] = xw[:, :h]
        xl_ref[:, h:] = jnp.ones((n, 16), jnp.float32)
        xr_ref[...] = xw[:, h:]

    return pl.pallas_call(
        body,
        out_shape=[jax.ShapeDtypeStruct((n, h + 16), jnp.float32),
                   jax.ShapeDtypeStruct((n, h), jnp.float32)],
    )(x, wcat)


def _mid(seg1p, xr, b1r, wcat2):
    n, h = xr.shape
    o2 = wcat2.shape[1]
    o = o2 // 2

    def body(s_ref, xr_ref, b1_ref, w_ref, zl_ref, zr_ref, inv_ref):
        sp = s_ref[...]
        deg = sp[0, :, h:h + 1] + sp[1, :, h:h + 1]
        inv = 1.0 / jnp.maximum(deg, 1.0)
        seg = sp[0, :, :h] + sp[1, :, :h]
        z1 = jnp.maximum(seg * inv + b1_ref[...] + xr_ref[...], 0.0)
        zw = jnp.dot(z1, w_ref[...], preferred_element_type=jnp.float32)
        zl_ref[...] = zw[:, :o]
        zr_ref[...] = zw[:, o:]
        inv_ref[...] = inv

    return pl.pallas_call(
        body,
        out_shape=[jax.ShapeDtypeStruct((n, o), jnp.float32),
                   jax.ShapeDtypeStruct((n, o), jnp.float32),
                   jax.ShapeDtypeStruct((n, 1), jnp.float32)],
    )(seg1p, xr, b1r, wcat2)


def _final_nodes(seg2p, inv, zr, b2r):
    n, o = zr.shape

    def body(s_ref, i_ref, zr_ref, b2_ref, z_ref):
        sp = s_ref[...]
        z_ref[...] = (sp[0] + sp[1]) * i_ref[...] + b2_ref[...] + zr_ref[...]

    return pl.pallas_call(
        body,
        out_shape=jax.ShapeDtypeStruct((n, o), jnp.float32),
    )(seg2p, inv, zr, b2r)


def _mlp(zu, zv, pfp, w1s, wpf, b1m, w2t, b2m, w3p, b3p, bp=2048):
    pp, o = zu.shape
    pfd = pfp.shape[1]
    mh = w1s.shape[1]
    mh2 = w2t.shape[1]
    ow = w3p.shape[1]
    grid = pp // bp

    def body(zu_ref, zv_ref, pf_ref, w1_ref, wp_ref, b1_ref, w2_ref, b2_ref,
             w3_ref, b3_ref, out_ref):
        a = zu_ref[...]
        bv = zv_ref[...]
        ad = jnp.abs(a - bv)
        pr = a * bv
        h1 = (jnp.dot(a, w1_ref[0:o], preferred_element_type=jnp.float32)
              + jnp.dot(bv, w1_ref[o:2 * o], preferred_element_type=jnp.float32)
              + jnp.dot(ad, w1_ref[2 * o:3 * o], preferred_element_type=jnp.float32)
              + jnp.dot(pr, w1_ref[3 * o:4 * o], preferred_element_type=jnp.float32)
              + jnp.dot(pf_ref[...], wp_ref[...], preferred_element_type=jnp.float32)
              + b1_ref[...])
        h1 = jnp.maximum(h1, 0.0)
        h2 = jnp.maximum(jnp.dot(h1, w2_ref[...], preferred_element_type=jnp.float32)
                         + b2_ref[...], 0.0)
        out_ref[...] = jnp.dot(h2, w3_ref[...], preferred_element_type=jnp.float32) + b3_ref[...]

    return pl.pallas_call(
        body,
        grid=(grid,),
        in_specs=[
            pl.BlockSpec((bp, o), lambda i: (i, 0)),
            pl.BlockSpec((bp, o), lambda i: (i, 0)),
            pl.BlockSpec((bp, pfd), lambda i: (i, 0)),
            pl.BlockSpec((4 * o, mh), lambda i: (0, 0)),
            pl.BlockSpec((pfd, mh), lambda i: (0, 0)),
            pl.BlockSpec((1, mh), lambda i: (0, 0)),
            pl.BlockSpec((mh, mh2), lambda i: (0, 0)),
            pl.BlockSpec((1, mh2), lambda i: (0, 0)),
            pl.BlockSpec((mh2, ow), lambda i: (0, 0)),
            pl.BlockSpec((1, ow), lambda i: (0, 0)),
        ],
        out_specs=pl.BlockSpec((bp, ow), lambda i: (i, 0)),
        out_shape=jax.ShapeDtypeStruct((pp, ow), jnp.float32),
    )(zu, zv, pfp, w1s, wpf, b1m, w2t, b2m, w3p, b3p)


def kernel(x, edge_index, edge_label_index, pair_feats,
           Wl1, Wr1, b1, Wl2, Wr2, b2, Wm1, bm1, Wm2, bm2, Wm3, bm3):
    n = x.shape[0]
    e = edge_index.shape[1]
    p = edge_label_index.shape[1]
    h = Wl1.shape[0]
    o = Wl2.shape[0]
    mh = Wm1.shape[0]
    mh2 = Wm2.shape[0]
    pfd = pair_feats.shape[1]

    nunit = _NS * 8
    n_pad = ((n + nunit) // nunit) * nunit
    xp = jnp.pad(x, ((0, n_pad - n), (0, 0)))

    src3, _ = _shard_indices(edge_index[0], n, n_pad - n)
    dst3, _ = _shard_indices(edge_index[1], n, n_pad - n)

    wcat1 = jnp.concatenate([Wl1.T, Wr1.T], axis=1)
    xlaug, xr = _encode_in(xp, wcat1)

    zeros_h = jnp.zeros((n_pad, h + 16), jnp.float32)
    seg1p = _seg_sum_sc(xlaug, src3, dst3, zeros_h)

    wcat2 = jnp.concatenate([Wl2.T, Wr2.T], axis=1)
    zl, zr, inv = _mid(seg1p, xr, b1.reshape(1, h), wcat2)

    zeros_o = jnp.zeros((n_pad, o), jnp.float32)
    seg2p = _seg_sum_sc(zl, src3, dst3, zeros_o)

    z = _final_nodes(seg2p, inv, zr, b2.reshape(1, o))

    u3, ppw_real = _shard_indices(edge_label_index[0], 0, n)
    v3, _ = _shard_indices(edge_label_index[1], 0, n)
    steps_p = u3.shape[1]
    ppw_pad = steps_p * _B
    pp = _NW * ppw_pad
    zu, zv = _pair_gather_sc(z, u3, v3, pp)

    pfpad = 8
    pfw = jnp.pad(pair_feats, ((0, _NW * ppw_real - p), (0, pfpad - pfd)))
    pfw = pfw.reshape(_NW, ppw_real, pfpad)
    pfp = jnp.pad(pfw, ((0, 0), (0, ppw_pad - ppw_real), (0, 0)))
    pfp = pfp.reshape(pp, pfpad)
    w1s = Wm1.T[:4 * o]
    wpf = jnp.pad(Wm1.T[4 * o:], ((0, pfpad - pfd), (0, 0)))
    w3p = jnp.pad(Wm3.T, ((0, 0), (0, 7)))
    b3p = jnp.pad(bm3.reshape(1, 1), ((0, 0), (0, 7)))
    out8 = _mlp(zu, zv, pfp, w1s, wpf, bm1.reshape(1, mh), Wm2.T,
                bm2.reshape(1, mh2), w3p, b3p)
    out = out8[:, 0].reshape(_NW, ppw_pad)[:, :ppw_real].reshape(-1)
    return out[:p]
